# baseline (speedup 1.0000x reference)
import functools

import jax
import jax.numpy as jnp
from jax import lax
from jax.experimental import pallas as pl
from jax.experimental.pallas import tpu as pltpu
from jax.experimental.pallas import tpu_sc as plsc

N = 50000
E = 3200000
D = 32
DE = 8

BE = 3200
BN = 5000

_f32 = jnp.float32


def _full(shape):
    return pl.BlockSpec(shape, lambda i: tuple(0 for _ in shape))


def _rows(block_rows, cols):
    return pl.BlockSpec((block_rows, cols), lambda i: (i, 0))


def _edge_u2a_body(xs, attr, we0, weA, be, wm0, wmE, bm, w3E, b3, e_o, m_o, h_o):
    x = xs[...]
    a = attr[...]
    e = jnp.maximum(x * we0[...] + jnp.dot(a, weA[...], preferred_element_type=_f32)
                    + be[...], 0.0)
    e_o[...] = e
    m_o[...] = jnp.maximum(x * wm0[...] + jnp.dot(e, wmE[...], preferred_element_type=_f32)
                           + bm[...], 0.0)
    h_o[...] = jnp.dot(e, w3E[...], preferred_element_type=_f32) + b3[...]


def _edge_a2u_body(attr, we, be, wm, bm, w3E, b3, e_o, m_o, h_o):
    a = attr[...]
    e = jnp.maximum(jnp.dot(a, we[...], preferred_element_type=_f32) + be[...], 0.0)
    e_o[...] = e
    m_o[...] = jnp.maximum(jnp.dot(e, wm[...], preferred_element_type=_f32) + bm[...], 0.0)
    h_o[...] = jnp.dot(e, w3E[...], preferred_element_type=_f32) + b3[...]


def _node_ap1_body(agg, wu, bu, w3, x_o, g_o):
    x = jnp.maximum(jnp.dot(agg[...], wu[...], preferred_element_type=_f32) + bu[...], 0.0)
    x_o[...] = x
    g_o[...] = jnp.dot(x, w3[...], preferred_element_type=_f32)


def _node_ue1_body(agg, xu, wuA, wu0, bu, w3, x_o, g_o):
    x = jnp.maximum(jnp.dot(agg[...], wuA[...], preferred_element_type=_f32)
                    + xu[...] * wu0[...] + bu[...], 0.0)
    x_o[...] = x
    g_o[...] = jnp.dot(x, w3[...], preferred_element_type=_f32)


def _node_ap2_body(agg, xp, wA, wB, b, x_o):
    x_o[...] = jnp.maximum(jnp.dot(agg[...], wA[...], preferred_element_type=_f32)
                           + jnp.dot(xp[...], wB[...], preferred_element_type=_f32)
                           + b[...], 0.0)


def _node_ue2_body(agg, xp, wA, wB, b, wp1, bp1, wp2, bp2, o_o):
    x = jnp.maximum(jnp.dot(agg[...], wA[...], preferred_element_type=_f32)
                    + jnp.dot(xp[...], wB[...], preferred_element_type=_f32)
                    + b[...], 0.0)
    p = jnp.maximum(jnp.dot(x, wp1[...], preferred_element_type=_f32) + bp1[...], 0.0)
    p = jax.nn.sigmoid(jnp.dot(p, wp2[...], preferred_element_type=_f32) + bp2[...])
    o_o[...] = jnp.concatenate([x[:, :1], p], axis=1)


def _tc_call(body, grid, in_specs, out_specs, out_shapes):
    return pl.pallas_call(
        body, grid=grid, in_specs=in_specs, out_specs=out_specs,
        out_shape=out_shapes)


CH = 1024
NCHUNK = E // CH
NW = 32
N_EXTRA = NCHUNK % NW
N_BASE = NCHUNK // NW
N_ACC = 50176
ROWS_T = N_ACC // 16

_sc_mesh = plsc.VectorSubcoreMesh(core_axis_name="c", subcore_axis_name="s")


def _worker_id():
    return lax.axis_index("c") * 16 + lax.axis_index("s")


def _n_chunks(wid):
    return N_BASE + jnp.where(wid < N_EXTRA, 1, 0)


def _sc_gather_x_body(tab_hbm, idx_hbm, out_hbm, idx_v, out_v):
    wid = _worker_id()

    def body(i, carry):
        ci = wid + i * NW
        pltpu.sync_copy(idx_hbm.at[pl.ds(ci * 8, 8)], idx_v)
        for j in range(8):
            pltpu.sync_copy(tab_hbm.at[idx_v.at[j]], out_v.at[j])
        pltpu.sync_copy(out_v, out_hbm.at[pl.ds(ci * 8, 8)])
        return carry

    lax.fori_loop(0, _n_chunks(wid), body, 0)


@functools.partial(
    pl.kernel,
    out_type=jax.ShapeDtypeStruct((E // 128, 128), _f32),
    mesh=_sc_mesh,
    scratch_types=[
        pltpu.VMEM((8, 128), jnp.int32),
        pltpu.VMEM((8, 128), _f32),
    ],
)
def _sc_gather_x(tab_hbm, idx_hbm, out_hbm, idx_v, out_v):
    _sc_gather_x_body(tab_hbm, idx_hbm, out_hbm, idx_v, out_v)


def _sc_scatter_body(m_hbm, idx_hbm, z_hbm, out_hbm, idx_v, m_v, acc):
    c = lax.axis_index("c")
    s = lax.axis_index("s")
    wid = c * 16 + s
    r0 = s * ROWS_T
    pltpu.sync_copy(z_hbm.at[pl.ds(r0, ROWS_T)], acc.at[pl.ds(r0, ROWS_T)])
    plsc.subcore_barrier()

    def body(i, carry):
        ci = wid + i * NW
        pltpu.sync_copy(idx_hbm.at[pl.ds(ci * 8, 8)], idx_v)
        pltpu.sync_copy(m_hbm.at[pl.ds(ci * CH, CH)], m_v)
        for j in range(8):
            pltpu.sync_copy(m_v.at[pl.ds(j * 128, 128)], acc.at[idx_v.at[j]],
                            add=True)
        return carry

    lax.fori_loop(0, _n_chunks(wid), body, 0)
    plsc.subcore_barrier()
    pltpu.sync_copy(acc.at[pl.ds(r0, ROWS_T)],
                    out_hbm.at[pl.ds(c * N_ACC + r0, ROWS_T)])


@functools.partial(
    pl.kernel,
    out_type=jax.ShapeDtypeStruct((2 * N_ACC, D), _f32),
    mesh=_sc_mesh,
    scratch_types=[
        pltpu.VMEM((8, 128), jnp.int32),
        pltpu.VMEM((CH, D), _f32),
        pltpu.VMEM_SHARED((N_ACC, D), _f32),
    ],
)
def _sc_scatter(m_hbm, idx_hbm, z_hbm, out_hbm, idx_v, m_v, acc):
    _sc_scatter_body(m_hbm, idx_hbm, z_hbm, out_hbm, idx_v, m_v, acc)


def _sc_fused_body(g_hbm, h_hbm, src_hbm, dst_hbm, z_hbm, out_hbm,
                   srcw_v, dstw_v, g_v, h_v, acc, sem):
    c = lax.axis_index("c")
    s = lax.axis_index("s")
    wid = c * 16 + s
    r0 = s * ROWS_T
    pltpu.sync_copy(z_hbm.at[pl.ds(r0, ROWS_T)], acc.at[pl.ds(r0, ROWS_T)])
    plsc.subcore_barrier()

    def body(i, carry):
        ci = wid + i * NW
        pltpu.sync_copy(src_hbm.at[pl.ds(ci * 8, 8)], srcw_v)
        pltpu.sync_copy(dst_hbm.at[pl.ds(ci * 8, 8)], dstw_v)
        pltpu.sync_copy(h_hbm.at[pl.ds(ci * CH, CH)], h_v)
        for j in range(8):
            pltpu.async_copy(g_hbm.at[srcw_v.at[j]],
                             g_v.at[pl.ds(j * 128, 128)], sem).wait()

        def rowfn(r, carry2):
            for half in range(2):
                col = half * 16
                v = g_v[r, pl.ds(col, 16)] + h_v[r, pl.ds(col, 16)]
                g_v[r, pl.ds(col, 16)] = jnp.maximum(v, 0.0)
            return carry2

        lax.fori_loop(0, CH, rowfn, 0)
        for j in range(8):
            pltpu.sync_copy(g_v.at[pl.ds(j * 128, 128)], acc.at[dstw_v.at[j]],
                            add=True)
        return carry

    lax.fori_loop(0, _n_chunks(wid), body, 0)
    plsc.subcore_barrier()
    pltpu.sync_copy(acc.at[pl.ds(r0, ROWS_T)],
                    out_hbm.at[pl.ds(c * N_ACC + r0, ROWS_T)])


@functools.partial(
    pl.kernel,
    out_type=jax.ShapeDtypeStruct((2 * N_ACC, D), _f32),
    mesh=_sc_mesh,
    scratch_types=[
        pltpu.VMEM((8, 128), jnp.int32),
        pltpu.VMEM((8, 128), jnp.int32),
        pltpu.VMEM((CH, D), _f32),
        pltpu.VMEM((CH, D), _f32),
        pltpu.VMEM_SHARED((N_ACC, D), _f32),
        pltpu.SemaphoreType.DMA,
    ],
)
def _sc_fused(g_hbm, h_hbm, src_hbm, dst_hbm, z_hbm, out_hbm,
              srcw_v, dstw_v, g_v, h_v, acc, sem):
    _sc_fused_body(g_hbm, h_hbm, src_hbm, dst_hbm, z_hbm, out_hbm,
                   srcw_v, dstw_v, g_v, h_v, acc, sem)


def kernel(x_ue, edge_index_u2a, edge_attr_u2a, edge_index_a2u, edge_attr_a2u,
           W_e_u2a, b_e_u2a, W_e_a2u, b_e_a2u, W_m_ap, b_m_ap, W_m_ue, b_m_ue,
           W_u_ap, b_u_ap, W_u_ue, b_u_ue,
           W3_m_ap, b3_m_ap, W3_m_ue, b3_m_ue, W3_u_ap, b3_u_ap, W3_u_ue, b3_u_ue,
           Wp1, bp1, Wp2, bp2):
    src_u = edge_index_u2a[0]
    dst_a = edge_index_u2a[1]
    src_a = edge_index_a2u[0]
    dst_u = edge_index_a2u[1]

    r2 = lambda b: b.reshape(1, -1)

    xs = jnp.take(x_ue, src_u, axis=0)

    e_u2a, m_ap, H1 = _tc_call(
        _edge_u2a_body, (E // BE,),
        [_rows(BE, 1), _rows(BE, 2),
         _full((1, DE)), _full((2, DE)), _full((1, DE)),
         _full((1, D)), _full((DE, D)), _full((1, D)),
         _full((DE, D)), _full((1, D))],
        [_rows(BE, DE), _rows(BE, D), _rows(BE, D)],
        [jax.ShapeDtypeStruct((E, DE), _f32),
         jax.ShapeDtypeStruct((E, D), _f32),
         jax.ShapeDtypeStruct((E, D), _f32)],
    )(xs, edge_attr_u2a,
      W_e_u2a[0:1], W_e_u2a[1:3], r2(b_e_u2a),
      W_m_ap[0:1], W_m_ap[1:], r2(b_m_ap),
      W3_m_ap[D:], r2(b3_m_ap))

    e_a2u, m_ue, H2 = _tc_call(
        _edge_a2u_body, (E // BE,),
        [_rows(BE, 2),
         _full((2, DE)), _full((1, DE)),
         _full((DE, D)), _full((1, D)),
         _full((DE, D)), _full((1, D))],
        [_rows(BE, DE), _rows(BE, D), _rows(BE, D)],
        [jax.ShapeDtypeStruct((E, DE), _f32),
         jax.ShapeDtypeStruct((E, D), _f32),
         jax.ShapeDtypeStruct((E, D), _f32)],
    )(edge_attr_a2u,
      W_e_a2u, r2(b_e_a2u),
      W_m_ue, r2(b_m_ue),
      W3_m_ue[D:], r2(b3_m_ue))

    agg_ap = jax.ops.segment_sum(m_ap, dst_a, num_segments=N)
    agg_ue = jax.ops.segment_sum(m_ue, dst_u, num_segments=N)

    x_ap1, G2 = _tc_call(
        _node_ap1_body, (N // BN,),
        [_rows(BN, D), _full((D, D)), _full((1, D)), _full((D, D))],
        [_rows(BN, D), _rows(BN, D)],
        [jax.ShapeDtypeStruct((N, D), _f32), jax.ShapeDtypeStruct((N, D), _f32)],
    )(agg_ap, W_u_ap, r2(b_u_ap), W3_m_ue[:D])

    x_ue1, G1 = _tc_call(
        _node_ue1_body, (N // BN,),
        [_rows(BN, D), _rows(BN, 1), _full((D, D)), _full((1, D)), _full((1, D)),
         _full((D, D))],
        [_rows(BN, D), _rows(BN, D)],
        [jax.ShapeDtypeStruct((N, D), _f32), jax.ShapeDtypeStruct((N, D), _f32)],
    )(agg_ue, x_ue, W_u_ue[:D], W_u_ue[D:], r2(b_u_ue), W3_m_ap[:D])

    m3_ap = jnp.maximum(jnp.take(G1, src_u, axis=0) + H1, 0.0)
    agg3_ap = jax.ops.segment_sum(m3_ap, dst_a, num_segments=N)
    m3_ue = jnp.maximum(jnp.take(G2, src_a, axis=0) + H2, 0.0)
    agg3_ue = jax.ops.segment_sum(m3_ue, dst_u, num_segments=N)

    x_ap2 = _tc_call(
        _node_ap2_body, (N // BN,),
        [_rows(BN, D), _rows(BN, D), _full((D, D)), _full((D, D)), _full((1, D))],
        _rows(BN, D),
        jax.ShapeDtypeStruct((N, D), _f32),
    )(agg3_ap, x_ap1, W3_u_ap[:D], W3_u_ap[D:], r2(b3_u_ap))

    out_ue = _tc_call(
        _node_ue2_body, (N // BN,),
        [_rows(BN, D), _rows(BN, D), _full((D, D)), _full((D, D)), _full((1, D)),
         _full((D, 16)), _full((1, 16)), _full((16, 1)), _full((1, 1))],
        _rows(BN, 2),
        jax.ShapeDtypeStruct((N, 2), _f32),
    )(agg3_ue, x_ue1, W3_u_ue[:D], W3_u_ue[D:], r2(b3_u_ue),
      Wp1, r2(bp1), Wp2, r2(bp2))

    return (out_ue, x_ap2, e_u2a, e_a2u)
